# Initial kernel scaffold; baseline (speedup 1.0000x reference)
import jax
import jax.numpy as jnp
from jax.experimental import pallas as pl


def kernel(proposals, objectness):
    raise NotImplementedError("write your pallas kernel here")


# speedup vs baseline: 13.1828x; 13.1828x over previous
import jax
import jax.numpy as jnp
from jax.experimental import pallas as pl

_R, _C = 160, 128
_N0 = 15000
_NV = 20000
_K = 1000
_NMS_T = 0.7
_MIN_SZ = 0.001
_IMG = 1024.0
_NEG = float("-inf")


def _rpn_kernel(x1r, y1r, x2r, y2r, obr, sor, b1r, b2r, b3r, b4r):
    idx = (jax.lax.broadcasted_iota(jnp.int32, (_R, _C), 0) * _C
           + jax.lax.broadcasted_iota(jnp.int32, (_R, _C), 1))
    ob = obr[...]
    lvl0 = idx < _N0
    lvl1 = (idx >= _N0) & (idx < _NV)

    def kth_threshold(mask):
        big = jnp.float32(3.0e38)
        lo0 = jnp.min(jnp.where(mask, ob, big))
        hi0 = jnp.max(jnp.where(mask, ob, -big)) + 1.0

        def body(_, lh):
            lo, hi = lh
            mid = 0.5 * (lo + hi)
            cnt = jnp.sum((mask & (ob >= mid)).astype(jnp.int32))
            take = cnt >= _K
            return (jnp.where(take, mid, lo), jnp.where(take, hi, mid))

        lo, hi = jax.lax.fori_loop(0, 64, body, (lo0, hi0))
        return lo

    th0 = kth_threshold(lvl0)
    th1 = kth_threshold(lvl1)
    sel = (lvl0 & (ob >= th0)) | (lvl1 & (ob >= th1))

    x1 = jnp.clip(x1r[...], 0.0, _IMG)
    y1 = jnp.clip(y1r[...], 0.0, _IMG)
    x2 = jnp.clip(x2r[...], 0.0, _IMG)
    y2 = jnp.clip(y2r[...], 0.0, _IMG)
    sc = jax.nn.sigmoid(ob)
    valid = sel & (x2 - x1 >= _MIN_SZ) & (y2 - y1 >= _MIN_SZ) & (sc >= 0.0)
    s0 = jnp.where(valid, sc, _NEG)

    off = jnp.where(idx >= _N0, 4096.0, 0.0)
    nx1 = x1 + off
    ny1 = y1 + off
    nx2 = x2 + off
    ny2 = y2 + off
    area = (nx2 - nx1) * (ny2 - ny1)

    oidx = (jax.lax.broadcasted_iota(jnp.int32, (8, _C), 0) * _C
            + jax.lax.broadcasted_iota(jnp.int32, (8, _C), 1))
    zacc = jnp.zeros((8, _C), jnp.float32)

    def cond(carry):
        t, m = carry[0], carry[1]
        return (t < _K) & (m > _NEG)

    def body(carry):
        t, m, s, so, a1, a2, a3, a4 = carry
        pidx = jnp.min(jnp.where(s == m, idx, jnp.int32(1 << 30)))
        hot = idx == pidx
        px1 = jnp.sum(jnp.where(hot, x1, 0.0))
        py1 = jnp.sum(jnp.where(hot, y1, 0.0))
        px2 = jnp.sum(jnp.where(hot, x2, 0.0))
        py2 = jnp.sum(jnp.where(hot, y2, 0.0))
        poff = jnp.where(pidx >= _N0, 4096.0, 0.0)
        qx1 = px1 + poff
        qy1 = py1 + poff
        qx2 = px2 + poff
        qy2 = py2 + poff
        parea = (qx2 - qx1) * (qy2 - qy1)
        iw = jnp.clip(jnp.minimum(qx2, nx2) - jnp.maximum(qx1, nx1), 0.0, None)
        ih = jnp.clip(jnp.minimum(qy2, ny2) - jnp.maximum(qy1, ny1), 0.0, None)
        inter = iw * ih
        iou = inter / jnp.maximum(parea + area - inter, 1e-9)
        s2 = jnp.where(iou > _NMS_T, _NEG, s)
        ohot = oidx == t
        so = so + jnp.where(ohot, m, 0.0)
        a1 = a1 + jnp.where(ohot, px1, 0.0)
        a2 = a2 + jnp.where(ohot, py1, 0.0)
        a3 = a3 + jnp.where(ohot, px2, 0.0)
        a4 = a4 + jnp.where(ohot, py2, 0.0)
        return (t + 1, jnp.max(s2), s2, so, a1, a2, a3, a4)

    init = (jnp.int32(0), jnp.max(s0), s0, zacc, zacc, zacc, zacc, zacc)
    _, _, _, so, a1, a2, a3, a4 = jax.lax.while_loop(cond, body, init)
    sor[...] = so
    b1r[...] = a1
    b2r[...] = a2
    b3r[...] = a3
    b4r[...] = a4


def kernel(proposals, objectness):
    p = proposals[0]
    ob = objectness[0]
    pad = jnp.zeros((_R * _C - _NV,), jnp.float32)

    def prep(v):
        return jnp.concatenate([v, pad]).reshape(_R, _C)

    outs = pl.pallas_call(
        _rpn_kernel,
        out_shape=[jax.ShapeDtypeStruct((8, _C), jnp.float32)] * 5,
    )(prep(p[:, 0]), prep(p[:, 1]), prep(p[:, 2]), prep(p[:, 3]), prep(ob))
    so, b1, b2, b3, b4 = [o.reshape(-1)[:_K] for o in outs]
    boxes = jnp.stack([b1, b2, b3, b4], axis=-1)
    return boxes, so


# speedup vs baseline: 14.1376x; 1.0724x over previous
import jax
import jax.numpy as jnp
from jax.experimental import pallas as pl

_R, _C = 160, 128
_N0 = 15000
_NV = 20000
_K = 1000
_NMS_T = 0.7
_MIN_SZ = 0.001
_IMG = 1024.0
_NEG = float("-inf")


def _rpn_kernel(x1r, y1r, x2r, y2r, obr, sor, b1r, b2r, b3r, b4r):
    idx = (jax.lax.broadcasted_iota(jnp.int32, (_R, _C), 0) * _C
           + jax.lax.broadcasted_iota(jnp.int32, (_R, _C), 1))
    ob = obr[...]
    lvl0 = idx < _N0
    lvl1 = (idx >= _N0) & (idx < _NV)

    big = jnp.float32(3.0e38)
    lo0i = jnp.min(jnp.where(lvl0, ob, big))
    hi0i = jnp.max(jnp.where(lvl0, ob, -big)) + 1.0
    lo1i = jnp.min(jnp.where(lvl1, ob, big))
    hi1i = jnp.max(jnp.where(lvl1, ob, -big)) + 1.0

    def bis(_, c):
        lo0, hi0, lo1, hi1 = c
        mid0 = 0.5 * (lo0 + hi0)
        mid1 = 0.5 * (lo1 + hi1)
        cnt0 = jnp.sum((lvl0 & (ob >= mid0)).astype(jnp.int32))
        cnt1 = jnp.sum((lvl1 & (ob >= mid1)).astype(jnp.int32))
        t0 = cnt0 >= _K
        t1 = cnt1 >= _K
        return (jnp.where(t0, mid0, lo0), jnp.where(t0, hi0, mid0),
                jnp.where(t1, mid1, lo1), jnp.where(t1, hi1, mid1))

    th0, _, th1, _ = jax.lax.fori_loop(0, 48, bis, (lo0i, hi0i, lo1i, hi1i))
    sel = (lvl0 & (ob >= th0)) | (lvl1 & (ob >= th1))

    x1 = jnp.clip(x1r[...], 0.0, _IMG)
    y1 = jnp.clip(y1r[...], 0.0, _IMG)
    x2 = jnp.clip(x2r[...], 0.0, _IMG)
    y2 = jnp.clip(y2r[...], 0.0, _IMG)
    sc = jax.nn.sigmoid(ob)
    valid = sel & (x2 - x1 >= _MIN_SZ) & (y2 - y1 >= _MIN_SZ) & (sc >= 0.0)
    s0 = jnp.where(valid, sc, _NEG)

    off = jnp.where(idx >= _N0, 4096.0, 0.0)
    nx1 = x1 + off
    ny1 = y1 + off
    nx2 = x2 + off
    ny2 = y2 + off
    area = (nx2 - nx1) * (ny2 - ny1)

    oidx = (jax.lax.broadcasted_iota(jnp.int32, (8, _C), 0) * _C
            + jax.lax.broadcasted_iota(jnp.int32, (8, _C), 1))
    zacc = jnp.zeros((8, _C), jnp.float32)

    def cond(carry):
        t, m = carry[0], carry[1]
        return (t < _K) & (m > _NEG)

    lane = jax.lax.broadcasted_iota(jnp.int32, (1, _C), 1)

    def body(carry):
        t, m, s, so, a1, a2, a3, a4 = carry
        pidx = jnp.min(jnp.where(s == m, idx, jnp.int32(1 << 30)))
        r = pidx // _C
        l = pidx % _C
        lhot = lane == l
        row = lambda ref: jnp.clip(ref[pl.ds(r, 1), :], 0.0, _IMG)
        px1 = jnp.sum(jnp.where(lhot, row(x1r), 0.0))
        py1 = jnp.sum(jnp.where(lhot, row(y1r), 0.0))
        px2 = jnp.sum(jnp.where(lhot, row(x2r), 0.0))
        py2 = jnp.sum(jnp.where(lhot, row(y2r), 0.0))
        poff = jnp.where(pidx >= _N0, 4096.0, 0.0)
        qx1 = px1 + poff
        qy1 = py1 + poff
        qx2 = px2 + poff
        qy2 = py2 + poff
        parea = (qx2 - qx1) * (qy2 - qy1)
        iw = jnp.clip(jnp.minimum(qx2, nx2) - jnp.maximum(qx1, nx1), 0.0, None)
        ih = jnp.clip(jnp.minimum(qy2, ny2) - jnp.maximum(qy1, ny1), 0.0, None)
        inter = iw * ih
        sup = inter > _NMS_T * jnp.maximum(parea + area - inter, 1e-9)
        s2 = jnp.where(sup, _NEG, s)
        ohot = oidx == t
        so = so + jnp.where(ohot, m, 0.0)
        a1 = a1 + jnp.where(ohot, px1, 0.0)
        a2 = a2 + jnp.where(ohot, py1, 0.0)
        a3 = a3 + jnp.where(ohot, px2, 0.0)
        a4 = a4 + jnp.where(ohot, py2, 0.0)
        return (t + 1, jnp.max(s2), s2, so, a1, a2, a3, a4)

    init = (jnp.int32(0), jnp.max(s0), s0, zacc, zacc, zacc, zacc, zacc)
    _, _, _, so, a1, a2, a3, a4 = jax.lax.while_loop(cond, body, init)
    sor[...] = so
    b1r[...] = a1
    b2r[...] = a2
    b3r[...] = a3
    b4r[...] = a4


def kernel(proposals, objectness):
    p = proposals[0]
    ob = objectness[0]
    pad = jnp.zeros((_R * _C - _NV,), jnp.float32)

    def prep(v):
        return jnp.concatenate([v, pad]).reshape(_R, _C)

    outs = pl.pallas_call(
        _rpn_kernel,
        out_shape=[jax.ShapeDtypeStruct((8, _C), jnp.float32)] * 5,
    )(prep(p[:, 0]), prep(p[:, 1]), prep(p[:, 2]), prep(p[:, 3]), prep(ob))
    so, b1, b2, b3, b4 = [o.reshape(-1)[:_K] for o in outs]
    boxes = jnp.stack([b1, b2, b3, b4], axis=-1)
    return boxes, so


# speedup vs baseline: 14.2601x; 1.0087x over previous
import jax
import jax.numpy as jnp
from jax.experimental import pallas as pl

_R, _C = 160, 128
_N0 = 15000
_NV = 20000
_K = 1000
_NMS_T = 0.7
_MIN_SZ = 0.001
_IMG = 1024.0
_NEG = float("-inf")


def _rpn_kernel(x1r, y1r, x2r, y2r, obr, sor, b1r, b2r, b3r, b4r):
    idx = (jax.lax.broadcasted_iota(jnp.int32, (_R, _C), 0) * _C
           + jax.lax.broadcasted_iota(jnp.int32, (_R, _C), 1))
    ob = obr[...]
    lvl0 = idx < _N0
    lvl1 = (idx >= _N0) & (idx < _NV)

    big = jnp.float32(3.0e38)
    lo0i = jnp.min(jnp.where(lvl0, ob, big))
    hi0i = jnp.max(jnp.where(lvl0, ob, -big)) + 1.0
    lo1i = jnp.min(jnp.where(lvl1, ob, big))
    hi1i = jnp.max(jnp.where(lvl1, ob, -big)) + 1.0

    def bis(_, c):
        lo0, hi0, lo1, hi1 = c
        mid0 = 0.5 * (lo0 + hi0)
        mid1 = 0.5 * (lo1 + hi1)
        cnt0 = jnp.sum((lvl0 & (ob >= mid0)).astype(jnp.int32))
        cnt1 = jnp.sum((lvl1 & (ob >= mid1)).astype(jnp.int32))
        t0 = cnt0 >= _K
        t1 = cnt1 >= _K
        return (jnp.where(t0, mid0, lo0), jnp.where(t0, hi0, mid0),
                jnp.where(t1, mid1, lo1), jnp.where(t1, hi1, mid1))

    th0, _, th1, _ = jax.lax.fori_loop(0, 48, bis, (lo0i, hi0i, lo1i, hi1i))
    sel = (lvl0 & (ob >= th0)) | (lvl1 & (ob >= th1))

    x1 = jnp.clip(x1r[...], 0.0, _IMG)
    y1 = jnp.clip(y1r[...], 0.0, _IMG)
    x2 = jnp.clip(x2r[...], 0.0, _IMG)
    y2 = jnp.clip(y2r[...], 0.0, _IMG)
    sc = jax.nn.sigmoid(ob)
    valid = sel & (x2 - x1 >= _MIN_SZ) & (y2 - y1 >= _MIN_SZ) & (sc >= 0.0)
    s0 = jnp.where(valid, sc, _NEG)

    off = jnp.where(idx >= _N0, 4096.0, 0.0)
    nx1 = x1 + off
    ny1 = y1 + off
    nx2 = x2 + off
    ny2 = y2 + off
    area = (nx2 - nx1) * (ny2 - ny1)
    t_area = _NMS_T * area

    oidx = (jax.lax.broadcasted_iota(jnp.int32, (8, _C), 0) * _C
            + jax.lax.broadcasted_iota(jnp.int32, (8, _C), 1))
    zacc = jnp.zeros((8, _C), jnp.float32)

    def cond(carry):
        t, m = carry[0], carry[1]
        return (t < _K) & (m > _NEG)

    lane = jax.lax.broadcasted_iota(jnp.int32, (1, _C), 1)

    def body(carry):
        t, m, s, so, a1, a2, a3, a4 = carry
        pidx = jnp.min(jnp.where(s == m, idx, jnp.int32(1 << 30)))
        r = pidx // _C
        l = pidx % _C
        lhot = lane == l
        row = lambda ref: jnp.clip(ref[pl.ds(r, 1), :], 0.0, _IMG)
        px1 = jnp.sum(jnp.where(lhot, row(x1r), 0.0))
        py1 = jnp.sum(jnp.where(lhot, row(y1r), 0.0))
        px2 = jnp.sum(jnp.where(lhot, row(x2r), 0.0))
        py2 = jnp.sum(jnp.where(lhot, row(y2r), 0.0))
        poff = jnp.where(pidx >= _N0, 4096.0, 0.0)
        qx1 = px1 + poff
        qy1 = py1 + poff
        qx2 = px2 + poff
        qy2 = py2 + poff
        parea = (qx2 - qx1) * (qy2 - qy1)
        iw = jnp.clip(jnp.minimum(qx2, nx2) - jnp.maximum(qx1, nx1), 0.0, None)
        ih = jnp.clip(jnp.minimum(qy2, ny2) - jnp.maximum(qy1, ny1), 0.0, None)
        inter = iw * ih
        sup = (1.0 + _NMS_T) * inter > t_area + _NMS_T * parea
        s2 = jnp.where(sup, _NEG, s)
        ohot = oidx == t
        so = so + jnp.where(ohot, m, 0.0)
        a1 = a1 + jnp.where(ohot, px1, 0.0)
        a2 = a2 + jnp.where(ohot, py1, 0.0)
        a3 = a3 + jnp.where(ohot, px2, 0.0)
        a4 = a4 + jnp.where(ohot, py2, 0.0)
        return (t + 1, jnp.max(s2), s2, so, a1, a2, a3, a4)

    init = (jnp.int32(0), jnp.max(s0), s0, zacc, zacc, zacc, zacc, zacc)
    _, _, _, so, a1, a2, a3, a4 = jax.lax.while_loop(cond, body, init)
    sor[...] = so
    b1r[...] = a1
    b2r[...] = a2
    b3r[...] = a3
    b4r[...] = a4


def kernel(proposals, objectness):
    p = proposals[0]
    ob = objectness[0]
    pad = jnp.zeros((_R * _C - _NV,), jnp.float32)

    def prep(v):
        return jnp.concatenate([v, pad]).reshape(_R, _C)

    outs = pl.pallas_call(
        _rpn_kernel,
        out_shape=[jax.ShapeDtypeStruct((8, _C), jnp.float32)] * 5,
    )(prep(p[:, 0]), prep(p[:, 1]), prep(p[:, 2]), prep(p[:, 3]), prep(ob))
    so, b1, b2, b3, b4 = [o.reshape(-1)[:_K] for o in outs]
    boxes = jnp.stack([b1, b2, b3, b4], axis=-1)
    return boxes, so


# speedup vs baseline: 16.6844x; 1.1700x over previous
import jax
import jax.numpy as jnp
from jax.experimental import pallas as pl

_R, _C = 160, 128
_N0 = 15000
_NV = 20000
_K = 1000
_NMS_T = 0.7
_MIN_SZ = 0.001
_IMG = 1024.0
_NEG = float("-inf")


def _rpn_kernel(x1r, y1r, x2r, y2r, obr, sor, b1r, b2r, b3r, b4r):
    idx = (jax.lax.broadcasted_iota(jnp.int32, (_R, _C), 0) * _C
           + jax.lax.broadcasted_iota(jnp.int32, (_R, _C), 1))
    ob = obr[...]
    lvl0 = idx < _N0
    lvl1 = (idx >= _N0) & (idx < _NV)

    big = jnp.float32(3.0e38)
    lo0i = jnp.min(jnp.where(lvl0, ob, big))
    hi0i = jnp.max(jnp.where(lvl0, ob, -big)) + 1.0
    lo1i = jnp.min(jnp.where(lvl1, ob, big))
    hi1i = jnp.max(jnp.where(lvl1, ob, -big)) + 1.0

    def bis(_, c):
        lo0, hi0, lo1, hi1 = c
        mid0 = 0.5 * (lo0 + hi0)
        mid1 = 0.5 * (lo1 + hi1)
        cnt0 = jnp.sum((lvl0 & (ob >= mid0)).astype(jnp.int32))
        cnt1 = jnp.sum((lvl1 & (ob >= mid1)).astype(jnp.int32))
        t0 = cnt0 >= _K
        t1 = cnt1 >= _K
        return (jnp.where(t0, mid0, lo0), jnp.where(t0, hi0, mid0),
                jnp.where(t1, mid1, lo1), jnp.where(t1, hi1, mid1))

    th0, _, th1, _ = jax.lax.fori_loop(0, 48, bis, (lo0i, hi0i, lo1i, hi1i))
    sel = (lvl0 & (ob >= th0)) | (lvl1 & (ob >= th1))

    x1 = jnp.clip(x1r[...], 0.0, _IMG)
    y1 = jnp.clip(y1r[...], 0.0, _IMG)
    x2 = jnp.clip(x2r[...], 0.0, _IMG)
    y2 = jnp.clip(y2r[...], 0.0, _IMG)
    sc = jax.nn.sigmoid(ob)
    valid = sel & (x2 - x1 >= _MIN_SZ) & (y2 - y1 >= _MIN_SZ) & (sc >= 0.0)
    s0 = jnp.where(valid, sc, _NEG)

    off = jnp.where(idx >= _N0, 4096.0, 0.0)
    nx1 = x1 + off
    ny1 = y1 + off
    nx2 = x2 + off
    ny2 = y2 + off
    area = (nx2 - nx1) * (ny2 - ny1)
    t_area = _NMS_T * area

    oidx = (jax.lax.broadcasted_iota(jnp.int32, (8, _C), 0) * _C
            + jax.lax.broadcasted_iota(jnp.int32, (8, _C), 1))
    zacc = jnp.zeros((8, _C), jnp.float32)

    def cond(carry):
        t, m = carry[0], carry[1]
        return (t < _K) & (m > _NEG)

    lane = jax.lax.broadcasted_iota(jnp.int32, (1, _C), 1)

    def extract(s_vec, m_val):
        pidx = jnp.min(jnp.where(s_vec == m_val, idx, jnp.int32(1 << 30)))
        pc = jnp.minimum(pidx, _NV - 1)
        r = pc // _C
        lhot = lane == (pc % _C)
        row = lambda ref: jnp.clip(ref[pl.ds(r, 1), :], 0.0, _IMG)
        px1 = jnp.sum(jnp.where(lhot, row(x1r), 0.0))
        py1 = jnp.sum(jnp.where(lhot, row(y1r), 0.0))
        px2 = jnp.sum(jnp.where(lhot, row(x2r), 0.0))
        py2 = jnp.sum(jnp.where(lhot, row(y2r), 0.0))
        poff = jnp.where(pc >= _N0, 4096.0, 0.0)
        qb = (px1 + poff, py1 + poff, px2 + poff, py2 + poff)
        parea = (qb[2] - qb[0]) * (qb[3] - qb[1])
        return pidx, (px1, py1, px2, py2), qb, parea

    def sup_mask(qb, parea):
        iw = jnp.clip(jnp.minimum(qb[2], nx2) - jnp.maximum(qb[0], nx1), 0.0, None)
        ih = jnp.clip(jnp.minimum(qb[3], ny2) - jnp.maximum(qb[1], ny1), 0.0, None)
        return (1.0 + _NMS_T) * (iw * ih) > t_area + _NMS_T * parea

    def body(carry):
        t, m, s, so, a1, a2, a3, a4 = carry
        pidx, b, qb, parea = extract(s, m)
        sup1 = sup_mask(qb, parea)
        s_e = jnp.where(idx == pidx, _NEG, s)
        m2 = jnp.max(s_e)
        _, b2, qb2, parea2 = extract(s_e, m2)
        iw12 = jnp.clip(jnp.minimum(qb[2], qb2[2]) - jnp.maximum(qb[0], qb2[0]), 0.0, None)
        ih12 = jnp.clip(jnp.minimum(qb[3], qb2[3]) - jnp.maximum(qb[1], qb2[1]), 0.0, None)
        sup12 = (1.0 + _NMS_T) * (iw12 * ih12) > _NMS_T * (parea + parea2)
        accept2 = (~sup12) & (m2 > _NEG) & (t + 1 < _K)
        sup2 = sup_mask(qb2, parea2) & accept2
        s2 = jnp.where(sup1 | sup2, _NEG, s)
        ohot1 = oidx == t
        ohot2 = (oidx == t + 1) & accept2
        so = so + jnp.where(ohot1, m, 0.0) + jnp.where(ohot2, m2, 0.0)
        a1 = a1 + jnp.where(ohot1, b[0], 0.0) + jnp.where(ohot2, b2[0], 0.0)
        a2 = a2 + jnp.where(ohot1, b[1], 0.0) + jnp.where(ohot2, b2[1], 0.0)
        a3 = a3 + jnp.where(ohot1, b[2], 0.0) + jnp.where(ohot2, b2[2], 0.0)
        a4 = a4 + jnp.where(ohot1, b[3], 0.0) + jnp.where(ohot2, b2[3], 0.0)
        return (t + 1 + accept2.astype(jnp.int32), jnp.max(s2), s2,
                so, a1, a2, a3, a4)

    init = (jnp.int32(0), jnp.max(s0), s0, zacc, zacc, zacc, zacc, zacc)
    _, _, _, so, a1, a2, a3, a4 = jax.lax.while_loop(cond, body, init)
    sor[...] = so
    b1r[...] = a1
    b2r[...] = a2
    b3r[...] = a3
    b4r[...] = a4


def kernel(proposals, objectness):
    p = proposals[0]
    ob = objectness[0]
    pad = jnp.zeros((_R * _C - _NV,), jnp.float32)

    def prep(v):
        return jnp.concatenate([v, pad]).reshape(_R, _C)

    outs = pl.pallas_call(
        _rpn_kernel,
        out_shape=[jax.ShapeDtypeStruct((8, _C), jnp.float32)] * 5,
    )(prep(p[:, 0]), prep(p[:, 1]), prep(p[:, 2]), prep(p[:, 3]), prep(ob))
    so, b1, b2, b3, b4 = [o.reshape(-1)[:_K] for o in outs]
    boxes = jnp.stack([b1, b2, b3, b4], axis=-1)
    return boxes, so
